# baseline (speedup 1.0000x reference)
import functools
import math

import jax
import jax.numpy as jnp
from jax import lax
from jax.experimental import pallas as pl
from jax.experimental.pallas import tpu as pltpu
from jax.experimental.pallas import tpu_sc as plsc

_B, _TPH, _TMEL, _H = 16, 1024, 2048, 256
_F0_BIN = 256
_F0_MIN, _F0_MAX = 50.0, 1100.0
_MEL_MIN = 1127.0 * math.log(1.0 + _F0_MIN / 700.0)
_MEL_MAX = 1127.0 * math.log(1.0 + _F0_MAX / 700.0)

_HROWS = _B * _TPH
_ROWS = _B * _TMEL

_NC, _NS, _L = 2, 16, 16
_NW = _NC * _NS
_RPW = _ROWS // _NW
_CH = 64
_NCH = _RPW // _CH


def _prep_body(mel2ph_ref, f0_ref, f0d_ref, gidx_ref, pidx_ref):
    m = mel2ph_ref[...]
    f0 = f0_ref[...]
    f0d = jnp.where(m == 0, 0.0, jnp.exp2(f0))
    f0d_ref[...] = f0d
    f0_mel = 1127.0 * jnp.log(1.0 + f0d / 700.0)
    f0_mel = jnp.where(
        f0_mel > 0,
        (f0_mel - _MEL_MIN) * (_F0_BIN - 2) / (_MEL_MAX - _MEL_MIN) + 1.0,
        f0_mel)
    f0_mel = jnp.where(f0_mel <= 1.0, 1.0, f0_mel)
    f0_mel = jnp.where(f0_mel > _F0_BIN - 1, float(_F0_BIN - 1), f0_mel)
    pidx_ref[...] = (f0_mel + 0.5).astype(jnp.int32)
    b = lax.broadcasted_iota(jnp.int32, m.shape, 0)
    gidx_ref[...] = b * _TPH + jnp.maximum(m - 1, 0)


def _sc_body(hub_ref, ptab_ref, gidx_ref, pidx_ref, out_ref,
             gi_v, pi_v, a_v, b_v, sem_a, sem_b):
    wid = lax.axis_index("s") * _NC + lax.axis_index("c")

    def chunk(i, carry):
        base = wid * _RPW + i * _CH
        pltpu.sync_copy(gidx_ref.at[pl.ds(base, _CH)], gi_v)
        pltpu.sync_copy(pidx_ref.at[pl.ds(base, _CH)], pi_v)
        cpa = pltpu.async_copy(hub_ref.at[gi_v], a_v, sem_a)
        cpb = pltpu.async_copy(ptab_ref.at[pi_v], b_v, sem_b)
        cpa.wait()
        cpb.wait()

        def add_row(r, c2):
            for k in range(_H // _L):
                sl = pl.ds(k * _L, _L)
                a_v[r, sl] = a_v[r, sl] + b_v[r, sl]
            return c2

        lax.fori_loop(0, _CH, add_row, 0)
        pltpu.sync_copy(a_v, out_ref.at[pl.ds(base, _CH)])
        return carry

    lax.fori_loop(0, _NCH, chunk, 0)


@functools.lru_cache(maxsize=None)
def _get_sc_call():
    return pl.kernel(
        _sc_body,
        out_type=jax.ShapeDtypeStruct((_ROWS, _H), jnp.float32),
        mesh=plsc.VectorSubcoreMesh(core_axis_name="c", subcore_axis_name="s"),
        scratch_types=[
            pltpu.VMEM((_CH,), jnp.int32),
            pltpu.VMEM((_CH,), jnp.int32),
            pltpu.VMEM((_CH, _H), jnp.float32),
            pltpu.VMEM((_CH, _H), jnp.float32),
            pltpu.SemaphoreType.DMA,
            pltpu.SemaphoreType.DMA,
        ],
    )


def _finish_body(dec_ref, mel_ref, spk_ref, out_ref):
    x = dec_ref[0]
    spk = spk_ref[0]
    mask = (mel_ref[0] > 0).astype(jnp.float32)
    out_ref[0] = jnp.transpose(x + spk, (1, 0)) * mask


def kernel(hubert, spk_embed, f0, pitch_embed, mel2ph):
    f0d, gidx, pidx = pl.pallas_call(
        _prep_body,
        out_shape=(
            jax.ShapeDtypeStruct((_B, _TMEL), jnp.float32),
            jax.ShapeDtypeStruct((_B, _TMEL), jnp.int32),
            jax.ShapeDtypeStruct((_B, _TMEL), jnp.int32),
        ),
    )(mel2ph, f0)

    dec = _get_sc_call()(
        hubert.reshape(_HROWS, _H),
        pitch_embed,
        gidx.reshape(_ROWS),
        pidx.reshape(_ROWS),
    )

    out = pl.pallas_call(
        _finish_body,
        grid=(_B,),
        in_specs=[
            pl.BlockSpec((1, _TMEL, _H), lambda b: (b, 0, 0)),
            pl.BlockSpec((1, 1, _TMEL), lambda b: (b, 0, 0)),
            pl.BlockSpec((1, 1, _H), lambda b: (b, 0, 0)),
        ],
        out_specs=pl.BlockSpec((1, _H, _TMEL), lambda b: (b, 0, 0)),
        out_shape=jax.ShapeDtypeStruct((_B, _H, _TMEL), jnp.float32),
    )(dec.reshape(_B, _TMEL, _H), mel2ph.reshape(_B, 1, _TMEL),
      spk_embed.reshape(_B, 1, _H))

    return out, f0d


# speedup vs baseline: 18.1062x; 18.1062x over previous
import functools
import math

import jax
import jax.numpy as jnp
from jax import lax
from jax.experimental import pallas as pl
from jax.experimental.pallas import tpu as pltpu
from jax.experimental.pallas import tpu_sc as plsc

_B, _TPH, _TMEL, _H = 16, 1024, 2048, 256
_F0_BIN = 256
_F0_MIN, _F0_MAX = 50.0, 1100.0
_MEL_MIN = 1127.0 * math.log(1.0 + _F0_MIN / 700.0)
_MEL_MAX = 1127.0 * math.log(1.0 + _F0_MAX / 700.0)

_HROWS = _B * _TPH
_ROWS = _B * _TMEL

_NC, _NS, _L = 2, 16, 16
_NW = _NC * _NS
_RPW = _ROWS // _NW
_CH = 128
_NCH = _RPW // _CH


def _prep_body(mel2ph_ref, f0_ref, f0d_ref, gidx_ref, pidx_ref):
    m = mel2ph_ref[...]
    f0 = f0_ref[...]
    f0d = jnp.where(m == 0, 0.0, jnp.exp2(f0))
    f0d_ref[...] = f0d
    f0_mel = 1127.0 * jnp.log(1.0 + f0d / 700.0)
    f0_mel = jnp.where(
        f0_mel > 0,
        (f0_mel - _MEL_MIN) * (_F0_BIN - 2) / (_MEL_MAX - _MEL_MIN) + 1.0,
        f0_mel)
    f0_mel = jnp.where(f0_mel <= 1.0, 1.0, f0_mel)
    f0_mel = jnp.where(f0_mel > _F0_BIN - 1, float(_F0_BIN - 1), f0_mel)
    pidx_ref[...] = (f0_mel + 0.5).astype(jnp.int32)
    b = lax.broadcasted_iota(jnp.int32, m.shape, 0)
    gidx_ref[...] = b * _TPH + jnp.maximum(m - 1, 0)


def _sc_body(hub_ref, gidx_ref, out_ref, gi_all, a0, a1,
             gsem0, gsem1, wsem0, wsem1):
    wid = lax.axis_index("s") * _NC + lax.axis_index("c")
    base = wid * _RPW
    pltpu.sync_copy(gidx_ref.at[pl.ds(base, _RPW)], gi_all)
    bufs, gsems, wsems = (a0, a1), (gsem0, gsem1), (wsem0, wsem1)
    gathers = [None, None]
    writes = [None, None]
    for i in range(_NCH):
        b = i % 2
        if writes[b] is not None:
            writes[b].wait()
        gathers[b] = pltpu.async_copy(
            hub_ref.at[gi_all.at[pl.ds(i * _CH, _CH)]], bufs[b], gsems[b])
        if i > 0:
            pb = (i - 1) % 2
            gathers[pb].wait()
            writes[pb] = pltpu.async_copy(
                bufs[pb], out_ref.at[pl.ds(base + (i - 1) * _CH, _CH)],
                wsems[pb])
    last = (_NCH - 1) % 2
    gathers[last].wait()
    pltpu.sync_copy(bufs[last], out_ref.at[pl.ds(base + (_NCH - 1) * _CH, _CH)])


@functools.lru_cache(maxsize=None)
def _get_sc_call():
    return pl.kernel(
        _sc_body,
        out_type=jax.ShapeDtypeStruct((_ROWS, _H), jnp.float32),
        mesh=plsc.VectorSubcoreMesh(core_axis_name="c", subcore_axis_name="s"),
        scratch_types=[
            pltpu.VMEM((_RPW,), jnp.int32),
            pltpu.VMEM((_CH, _H), jnp.float32),
            pltpu.VMEM((_CH, _H), jnp.float32),
            pltpu.SemaphoreType.DMA,
            pltpu.SemaphoreType.DMA,
            pltpu.SemaphoreType.DMA,
            pltpu.SemaphoreType.DMA,
        ],
    )


def _finish_body(dec_ref, mel_ref, pidx_ref, pe_ref, spk_ref, out_ref):
    x = dec_ref[0]
    spk = spk_ref[0]
    mask = (mel_ref[0] > 0).astype(jnp.float32)
    pidx = pidx_ref[0]
    onehot = (lax.broadcasted_iota(jnp.int32, (_F0_BIN, _TMEL), 0)
              == pidx).astype(jnp.float32)
    pitch_t = lax.dot_general(pe_ref[...], onehot, (((0,), (0,)), ((), ())),
                              preferred_element_type=jnp.float32)
    out_ref[0] = (jnp.transpose(x + spk, (1, 0)) + pitch_t) * mask


def kernel(hubert, spk_embed, f0, pitch_embed, mel2ph):
    f0d, gidx, pidx = pl.pallas_call(
        _prep_body,
        out_shape=(
            jax.ShapeDtypeStruct((_B, _TMEL), jnp.float32),
            jax.ShapeDtypeStruct((_B, _TMEL), jnp.int32),
            jax.ShapeDtypeStruct((_B, _TMEL), jnp.int32),
        ),
    )(mel2ph, f0)

    dec = _get_sc_call()(hubert.reshape(_HROWS, _H), gidx.reshape(_ROWS))

    out = pl.pallas_call(
        _finish_body,
        grid=(_B,),
        in_specs=[
            pl.BlockSpec((1, _TMEL, _H), lambda b: (b, 0, 0)),
            pl.BlockSpec((1, 1, _TMEL), lambda b: (b, 0, 0)),
            pl.BlockSpec((1, 1, _TMEL), lambda b: (b, 0, 0)),
            pl.BlockSpec((_F0_BIN, _H), lambda b: (0, 0)),
            pl.BlockSpec((1, 1, _H), lambda b: (b, 0, 0)),
        ],
        out_specs=pl.BlockSpec((1, _H, _TMEL), lambda b: (b, 0, 0)),
        out_shape=jax.ShapeDtypeStruct((_B, _H, _TMEL), jnp.float32),
    )(dec.reshape(_B, _TMEL, _H), mel2ph.reshape(_B, 1, _TMEL),
      pidx.reshape(_B, 1, _TMEL), pitch_embed[:_F0_BIN],
      spk_embed.reshape(_B, 1, _H))

    return out, f0d
